# Initial kernel scaffold; baseline (speedup 1.0000x reference)
import jax
import jax.numpy as jnp
from jax.experimental import pallas as pl


def kernel(support_set, support_labels, query_set):
    raise NotImplementedError("write your pallas kernel here")


# speedup vs baseline: 3.3280x; 3.3280x over previous
import functools

import jax
import jax.numpy as jnp
from jax import lax
from jax.experimental import pallas as pl
from jax.experimental.pallas import tpu as pltpu
from jax.experimental.pallas import tpu_sc as plsc

NUM_CLASSES = 64
D = 128
EPS = 1e-8

NC, NS = 2, 16
NW = NC * NS
AW = D + 16
ACC_WORDS = NUM_CLASSES * AW

R = 256
NBUF = 2


def _sc_segment_sums(support, labels, n_rows):
    nblk_tot = n_rows // R

    mesh = plsc.VectorSubcoreMesh(
        core_axis_name="c", subcore_axis_name="s",
        num_cores=NC, num_subcores=NS)

    @functools.partial(
        pl.kernel,
        out_type=jax.ShapeDtypeStruct((NW * ACC_WORDS,), jnp.float32),
        mesh=mesh,
        scratch_types=[
            pltpu.VMEM((NBUF, R, D), jnp.float32),
            pltpu.VMEM((NBUF, R), jnp.int32),
            pltpu.VMEM((ACC_WORDS,), jnp.float32),
            pltpu.SemaphoreType.DMA,
            pltpu.SemaphoreType.DMA,
            pltpu.SemaphoreType.DMA,
            pltpu.SemaphoreType.DMA,
        ],
    )
    def seg_kernel(sup_hbm, lbl_hbm, out_hbm, rows_v, lbls_v, acc_v, *sems):
        row_sems = sems[:NBUF]
        lbl_sems = sems[NBUF:]
        cid = lax.axis_index("c")
        sid = lax.axis_index("s")
        wid = sid * NC + cid
        lo = (wid * nblk_tot) // NW
        hi = ((wid + 1) * nblk_tot) // NW
        nblk = hi - lo

        zeros = jnp.zeros((16,), jnp.float32)
        ones = jnp.ones((16,), jnp.float32)

        def zero_body(i, _):
            acc_v[pl.ds(16 * i, 16)] = zeros
            return 0
        lax.fori_loop(0, ACC_WORDS // 16, zero_body, 0)

        def start(blk, b):
            pltpu.make_async_copy(
                sup_hbm.at[pl.ds(blk * R, R), :],
                rows_v.at[b], row_sems[b]).start()
            pltpu.make_async_copy(
                lbl_hbm.at[pl.ds(blk * R, R)],
                lbls_v.at[b], lbl_sems[b]).start()

        def wait(blk, b):
            pltpu.make_async_copy(
                sup_hbm.at[pl.ds(blk * R, R), :],
                rows_v.at[b], row_sems[b]).wait()
            pltpu.make_async_copy(
                lbl_hbm.at[pl.ds(blk * R, R)],
                lbls_v.at[b], lbl_sems[b]).wait()

        for b in range(NBUF):
            start(lo + b, b)

        def outer(g, _):
            for b in range(NBUF):
                blk = lo + g * NBUF + b

                @pl.when(blk < hi)
                def _(b=b, blk=blk):
                    wait(blk, b)

                    def grp_body(gi, _, _b=b):
                        lblv = lbls_v[_b, pl.ds(16 * gi, 16)]
                        for k in range(16):
                            off = lblv[k] * AW
                            row = 16 * gi + k
                            for j in range(D // 16):
                                plsc.addupdate(
                                    acc_v.at[pl.ds(off + 16 * j, 16)],
                                    rows_v[_b, row, pl.ds(16 * j, 16)])
                            plsc.addupdate(
                                acc_v.at[pl.ds(off + D, 16)], ones)
                        return 0
                    lax.fori_loop(0, R // 16, grp_body, 0)

                    nxt = blk + NBUF

                    @pl.when(nxt < hi)
                    def _():
                        start(nxt, b)
            return 0

        lax.fori_loop(0, (nblk + NBUF - 1) // NBUF, outer, 0)

        pltpu.sync_copy(acc_v, out_hbm.at[pl.ds(wid * ACC_WORDS, ACC_WORDS)])

    return seg_kernel(support, labels)


def _tc_finish(partials, q):
    def body(part_ref, q_ref, out_ref):
        tot = jnp.sum(part_ref[...], axis=0)
        sums = tot[:, :D]
        counts = tot[:, D:D + 1]
        protos = sums / jnp.maximum(counts, 1.0)
        p_norm = jnp.maximum(
            jnp.sqrt(jnp.sum(protos * protos, axis=1, keepdims=True)), EPS)
        pn = protos / p_norm
        qv = q_ref[...]
        q_norm = jnp.maximum(
            jnp.sqrt(jnp.sum(qv * qv, axis=1, keepdims=True)), EPS)
        qn = qv / q_norm
        sim = lax.dot_general(
            qn, pn, (((1,), (1,)), ((), ())),
            precision=lax.Precision.HIGHEST,
            preferred_element_type=jnp.float32)
        m = jnp.max(sim, axis=1, keepdims=True)
        ex = jnp.exp(sim - m)
        out_ref[...] = (sim - m) - jnp.log(jnp.sum(ex, axis=1, keepdims=True))

    return pl.pallas_call(
        body,
        out_shape=jax.ShapeDtypeStruct((q.shape[0], NUM_CLASSES), jnp.float32),
    )(partials, q)


def kernel(support_set, support_labels, query_set):
    n_rows = support_set.shape[0]
    labels = support_labels.astype(jnp.int32)
    flat = _sc_segment_sums(support_set, labels, n_rows)
    partials = flat.reshape(NW, NUM_CLASSES, AW)
    q = query_set.reshape(query_set.shape[0], D)
    return _tc_finish(partials, q)
